# baseline (speedup 1.0000x reference)
import functools

import jax
import jax.numpy as jnp
from jax import lax
from jax.experimental import pallas as pl
from jax.experimental.pallas import tpu as pltpu
from jax.experimental.pallas import tpu_sc as plsc

B, N, DIM_EMBED = 8, 576, 384
DIM = 32
HEADS = 2
K = 8192
DIM_MEM = 32

TOKENS = B * N
N_SPLITS = 2
T_HALF = TOKENS // N_SPLITS
T_BLK = 576

SC_CORES = 2
SC_SUBCORES = 16
SC_WORKERS = SC_CORES * SC_SUBCORES
GATHER_CHUNK = 72


def _tc_idx_body(x_ref, rp_ref, cb_ref, cbn_ref, idx0_ref, idx1_ref):
    xb = x_ref[...]
    for h, out_ref in ((0, idx0_ref), (1, idx1_ref)):
        rp = rp_ref[h]
        cb = cb_ref[h]
        xp = jnp.dot(xb, rp)
        xp2 = jnp.sum(xp * xp, axis=-1, keepdims=True)
        dotn = lax.dot_general(xp * -2.0, cb, (((1,), (1,)), ((), ())))
        d2 = xp2 + dotn + cbn_ref[h][None, :]
        am = jnp.argmin(d2, axis=-1).astype(jnp.int32)
        out_ref[...] = (am + h * K)[None, None, :]


def _tc_indices(xf, rand_proj, codebook, cbn, tokens):
    g = tokens // T_BLK
    idx_struct = jax.ShapeDtypeStruct((g, 1, T_BLK), jnp.int32)
    idx_spec = pl.BlockSpec((1, 1, T_BLK), lambda i: (i, 0, 0))
    return pl.pallas_call(
        _tc_idx_body,
        grid=(g,),
        in_specs=[
            pl.BlockSpec((T_BLK, DIM_EMBED), lambda i: (i, 0)),
            pl.BlockSpec((HEADS, DIM_EMBED, DIM), lambda i: (0, 0, 0)),
            pl.BlockSpec((HEADS, K, DIM), lambda i: (0, 0, 0)),
            pl.BlockSpec((HEADS, K), lambda i: (0, 0)),
        ],
        out_specs=[idx_spec, idx_spec],
        out_shape=[idx_struct, idx_struct],
    )(xf, rand_proj, codebook, cbn)


def _make_sc_gather_mean(tokens):
    tok_per_w = tokens // SC_WORKERS
    n_chunks = tok_per_w // GATHER_CHUNK

    @functools.partial(
        pl.kernel,
        mesh=plsc.VectorSubcoreMesh(core_axis_name="c", subcore_axis_name="s"),
        compiler_params=pltpu.CompilerParams(use_tc_tiling_on_sc=False),
        out_type=jax.ShapeDtypeStruct((tokens, DIM_MEM), jnp.float32),
        scratch_types=[
            pltpu.VMEM((n_chunks, GATHER_CHUNK), jnp.int32),
            pltpu.VMEM((n_chunks, GATHER_CHUNK), jnp.int32),
            pltpu.VMEM((tok_per_w, DIM_MEM), jnp.float32),
            pltpu.VMEM((tok_per_w, DIM_MEM), jnp.float32),
            pltpu.VMEM((tok_per_w, DIM_MEM), jnp.float32),
            pltpu.SemaphoreType.DMA,
        ],
    )
    def sc_gather_mean(vals_hbm, idx0_hbm, idx1_hbm, out_hbm,
                       i0_v, i1_v, r0_v, r1_v, o_v, sem):
        wid = lax.axis_index("s") * SC_CORES + lax.axis_index("c")
        base = wid * tok_per_w
        pltpu.sync_copy(idx0_hbm.at[wid], i0_v)
        pltpu.sync_copy(idx1_hbm.at[wid], i1_v)
        copies = []
        for j in range(n_chunks):
            sl = pl.ds(j * GATHER_CHUNK, GATHER_CHUNK)
            copies.append(pltpu.async_copy(vals_hbm.at[i0_v.at[j]], r0_v.at[sl], sem))
            copies.append(pltpu.async_copy(vals_hbm.at[i1_v.at[j]], r1_v.at[sl], sem))
        for c in copies:
            c.wait()

        def body(t, carry):
            for j in range(DIM_MEM // 16):
                sl = pl.ds(j * 16, 16)
                o_v[t, sl] = (r0_v[t, sl] + r1_v[t, sl]) * 0.5
            return carry

        lax.fori_loop(0, tok_per_w, body, 0)
        pltpu.sync_copy(o_v, out_hbm.at[pl.ds(base, tok_per_w)])

    return sc_gather_mean


def kernel(x, rand_proj, values, codebook):
    xf = x.reshape(TOKENS, DIM_EMBED)
    cbn = jnp.sum(codebook * codebook, axis=-1)
    vals_flat = values.reshape(HEADS * K, DIM_MEM)
    sc_gather = _make_sc_gather_mean(T_HALF)
    sc_shape = (SC_WORKERS, T_HALF // SC_WORKERS // GATHER_CHUNK, GATHER_CHUNK)
    outs = []
    for s in range(N_SPLITS):
        xh = lax.slice_in_dim(xf, s * T_HALF, (s + 1) * T_HALF, axis=0)
        idx0, idx1 = _tc_indices(xh, rand_proj, codebook, cbn, T_HALF)
        outs.append(sc_gather(vals_flat, idx0.reshape(sc_shape),
                              idx1.reshape(sc_shape)))
    out = jnp.concatenate(outs, axis=0)
    return out.reshape(B, N, DIM_MEM)


# speedup vs baseline: 1.1183x; 1.1183x over previous
import functools

import jax
import jax.numpy as jnp
from jax import lax
from jax.experimental import pallas as pl
from jax.experimental.pallas import tpu as pltpu
from jax.experimental.pallas import tpu_sc as plsc

B, N, DIM_EMBED = 8, 576, 384
DIM = 32
HEADS = 2
K = 8192
DIM_MEM = 32

TOKENS = B * N
N_SPLITS = 1
T_HALF = TOKENS // N_SPLITS
T_BLK = 256

SC_CORES = 2
SC_SUBCORES = 16
SC_WORKERS = SC_CORES * SC_SUBCORES
GATHER_CHUNK = 72


def _tc_idx_body(x_ref, rp_ref, cb_ref, cbn_ref, idx0_ref, idx1_ref):
    xb = x_ref[...]
    for h, out_ref in ((0, idx0_ref), (1, idx1_ref)):
        rp = rp_ref[h]
        cb = cb_ref[h]
        xp = jnp.dot(xb, rp)
        xp2 = jnp.sum(xp * xp, axis=-1, keepdims=True)
        dotn = lax.dot_general(xp * -2.0, cb, (((1,), (1,)), ((), ())))
        d2 = xp2 + dotn + cbn_ref[h][None, :]
        am = jnp.argmin(d2, axis=-1).astype(jnp.int32)
        out_ref[...] = (am + h * K)[None, None, :]


def _tc_indices(xf, rand_proj, codebook, cbn, tokens):
    g = tokens // T_BLK
    idx_struct = jax.ShapeDtypeStruct((g, 1, T_BLK), jnp.int32)
    idx_spec = pl.BlockSpec((1, 1, T_BLK), lambda i: (i, 0, 0))
    return pl.pallas_call(
        _tc_idx_body,
        grid=(g,),
        in_specs=[
            pl.BlockSpec((T_BLK, DIM_EMBED), lambda i: (i, 0)),
            pl.BlockSpec((HEADS, DIM_EMBED, DIM), lambda i: (0, 0, 0)),
            pl.BlockSpec((HEADS, K, DIM), lambda i: (0, 0, 0)),
            pl.BlockSpec((HEADS, K), lambda i: (0, 0)),
        ],
        out_specs=[idx_spec, idx_spec],
        out_shape=[idx_struct, idx_struct],
    )(xf, rand_proj, codebook, cbn)


def _make_sc_gather_mean(tokens):
    tok_per_w = tokens // SC_WORKERS
    n_chunks = tok_per_w // GATHER_CHUNK

    @functools.partial(
        pl.kernel,
        mesh=plsc.VectorSubcoreMesh(core_axis_name="c", subcore_axis_name="s"),
        compiler_params=pltpu.CompilerParams(use_tc_tiling_on_sc=False),
        out_type=jax.ShapeDtypeStruct((tokens, DIM_MEM), jnp.float32),
        scratch_types=[
            pltpu.VMEM((n_chunks, GATHER_CHUNK), jnp.int32),
            pltpu.VMEM((n_chunks, GATHER_CHUNK), jnp.int32),
            pltpu.VMEM((tok_per_w, DIM_MEM), jnp.float32),
            pltpu.VMEM((tok_per_w, DIM_MEM), jnp.float32),
            pltpu.SemaphoreType.DMA,
        ],
    )
    def sc_gather_mean(vals_hbm, idx0_hbm, idx1_hbm, out_hbm,
                       i0_v, i1_v, r0_v, r1_v, sem):
        wid = lax.axis_index("s") * SC_CORES + lax.axis_index("c")
        base = wid * tok_per_w
        pltpu.sync_copy(idx0_hbm.at[wid], i0_v)
        pltpu.sync_copy(idx1_hbm.at[wid], i1_v)
        copies = []
        for j in range(n_chunks):
            sl = pl.ds(j * GATHER_CHUNK, GATHER_CHUNK)
            copies.append(pltpu.async_copy(vals_hbm.at[i0_v.at[j]], r0_v.at[sl], sem))
            copies.append(pltpu.async_copy(vals_hbm.at[i1_v.at[j]], r1_v.at[sl], sem))
        for c in copies:
            c.wait()

        def body(t, carry):
            for u in range(4):
                for j in range(DIM_MEM // 16):
                    sl = pl.ds(j * 16, 16)
                    r0_v[t * 4 + u, sl] = (r0_v[t * 4 + u, sl]
                                           + r1_v[t * 4 + u, sl]) * 0.5
            return carry

        lax.fori_loop(0, tok_per_w // 4, body, 0)
        pltpu.sync_copy(r0_v, out_hbm.at[pl.ds(base, tok_per_w)])

    return sc_gather_mean


def kernel(x, rand_proj, values, codebook):
    xf = x.reshape(TOKENS, DIM_EMBED)
    cbn = jnp.sum(codebook * codebook, axis=-1)
    vals_flat = values.reshape(HEADS * K, DIM_MEM)
    sc_gather = _make_sc_gather_mean(T_HALF)
    sc_shape = (SC_WORKERS, T_HALF // SC_WORKERS // GATHER_CHUNK, GATHER_CHUNK)
    idx0, idx1 = _tc_indices(xf, rand_proj, codebook, cbn, T_HALF)
    out = sc_gather(vals_flat, idx0.reshape(sc_shape), idx1.reshape(sc_shape))
    return out.reshape(B, N, DIM_MEM)


# speedup vs baseline: 1.1437x; 1.0227x over previous
import functools

import jax
import jax.numpy as jnp
from jax import lax
from jax.experimental import pallas as pl
from jax.experimental.pallas import tpu as pltpu
from jax.experimental.pallas import tpu_sc as plsc

B, N, DIM_EMBED = 8, 576, 384
DIM = 32
HEADS = 2
K = 8192
DIM_MEM = 32

TOKENS = B * N
N_SPLITS = 1
T_HALF = TOKENS // N_SPLITS
T_BLK = 256

SC_CORES = 2
SC_SUBCORES = 16
SC_WORKERS = SC_CORES * SC_SUBCORES
GATHER_CHUNK = 72


def _tc_idx_body(x_ref, rp_ref, cb_ref, cbn_ref, idx_ref):
    xb = x_ref[...]
    outs = []
    for h in range(HEADS):
        rp = rp_ref[h]
        cb = cb_ref[h]
        xp = jnp.dot(xb, rp)
        xp2 = jnp.sum(xp * xp, axis=-1, keepdims=True)
        dotn = lax.dot_general(xp * -2.0, cb, (((1,), (1,)), ((), ())))
        d2 = xp2 + dotn + cbn_ref[h][None, :]
        am = jnp.argmin(d2, axis=-1).astype(jnp.int32)
        outs.append(am + h * K)
    idx_ref[...] = jnp.stack(outs)[None]


def _tc_indices(xf, rand_proj, codebook, cbn, tokens):
    g = tokens // T_BLK
    return pl.pallas_call(
        _tc_idx_body,
        grid=(g,),
        in_specs=[
            pl.BlockSpec((T_BLK, DIM_EMBED), lambda i: (i, 0)),
            pl.BlockSpec((HEADS, DIM_EMBED, DIM), lambda i: (0, 0, 0)),
            pl.BlockSpec((HEADS, K, DIM), lambda i: (0, 0, 0)),
            pl.BlockSpec((HEADS, K), lambda i: (0, 0)),
        ],
        out_specs=pl.BlockSpec((1, HEADS, T_BLK), lambda i: (i, 0, 0)),
        out_shape=jax.ShapeDtypeStruct((g, HEADS, T_BLK), jnp.int32),
    )(xf, rand_proj, codebook, cbn)


def _make_sc_gather_mean(tokens):
    tok_per_w = tokens // SC_WORKERS
    n_chunks = tok_per_w // GATHER_CHUNK

    @functools.partial(
        pl.kernel,
        mesh=plsc.VectorSubcoreMesh(core_axis_name="c", subcore_axis_name="s"),
        compiler_params=pltpu.CompilerParams(use_tc_tiling_on_sc=False),
        out_type=jax.ShapeDtypeStruct((tokens, DIM_MEM), jnp.float32),
        scratch_types=[
            pltpu.VMEM((n_chunks, GATHER_CHUNK), jnp.int32),
            pltpu.VMEM((n_chunks, GATHER_CHUNK), jnp.int32),
            pltpu.VMEM((tok_per_w, DIM_MEM), jnp.float32),
            pltpu.VMEM((tok_per_w, DIM_MEM), jnp.float32),
            pltpu.SemaphoreType.DMA,
        ],
    )
    def sc_gather_mean(vals_hbm, idx0_hbm, idx1_hbm, out_hbm,
                       i0_v, i1_v, r0_v, r1_v, sem):
        wid = lax.axis_index("s") * SC_CORES + lax.axis_index("c")
        base = wid * tok_per_w
        pltpu.sync_copy(idx0_hbm.at[wid], i0_v)
        pltpu.sync_copy(idx1_hbm.at[wid], i1_v)
        copies = []
        for j in range(n_chunks):
            sl = pl.ds(j * GATHER_CHUNK, GATHER_CHUNK)
            copies.append(pltpu.async_copy(vals_hbm.at[i0_v.at[j]], r0_v.at[sl], sem))
            copies.append(pltpu.async_copy(vals_hbm.at[i1_v.at[j]], r1_v.at[sl], sem))
        for c in copies:
            c.wait()

        def body(t, carry):
            for u in range(4):
                for j in range(DIM_MEM // 16):
                    sl = pl.ds(j * 16, 16)
                    r0_v[t * 4 + u, sl] = (r0_v[t * 4 + u, sl]
                                           + r1_v[t * 4 + u, sl]) * 0.5
            return carry

        lax.fori_loop(0, tok_per_w // 4, body, 0)
        pltpu.sync_copy(r0_v, out_hbm.at[pl.ds(base, tok_per_w)])

    return sc_gather_mean


def kernel(x, rand_proj, values, codebook):
    xf = x.reshape(TOKENS, DIM_EMBED)
    cbn = jnp.sum(codebook * codebook, axis=-1)
    vals_flat = values.reshape(HEADS * K, DIM_MEM)
    sc_gather = _make_sc_gather_mean(T_HALF)
    sc_shape = (SC_WORKERS, T_HALF // SC_WORKERS // GATHER_CHUNK, GATHER_CHUNK)
    idx = _tc_indices(xf, rand_proj, codebook, cbn, T_HALF)
    idxh = idx.transpose(1, 0, 2).reshape((HEADS,) + sc_shape)
    out = sc_gather(vals_flat, idxh[0], idxh[1])
    return out.reshape(B, N, DIM_MEM)


# speedup vs baseline: 1.1483x; 1.0040x over previous
import functools

import jax
import jax.numpy as jnp
from jax import lax
from jax.experimental import pallas as pl
from jax.experimental.pallas import tpu as pltpu
from jax.experimental.pallas import tpu_sc as plsc

B, N, DIM_EMBED = 8, 576, 384
DIM = 32
HEADS = 2
K = 8192
DIM_MEM = 32

TOKENS = B * N
N_SPLITS = 1
T_HALF = TOKENS // N_SPLITS
T_BLK = 256

SC_CORES = 2
SC_SUBCORES = 16
SC_WORKERS = SC_CORES * SC_SUBCORES
GATHER_CHUNK = 72


def _tc_idx_body(x_ref, rp_ref, cb_ref, cbn_ref, idx_ref):
    xb = x_ref[...]
    outs = []
    for h in range(HEADS):
        rp = rp_ref[h]
        cb = cb_ref[h]
        xp = jnp.dot(xb, rp)
        xp2 = jnp.sum(xp * xp, axis=-1, keepdims=True)
        dotn = lax.dot_general(xp * -2.0, cb, (((1,), (1,)), ((), ())))
        d2 = xp2 + dotn + cbn_ref[h][None, :]
        am = jnp.argmin(d2, axis=-1).astype(jnp.int32)
        outs.append(am + h * K)
    idx_ref[...] = jnp.stack(outs)[None]


def _tc_indices(xf, rand_proj, codebook, cbn, tokens):
    g = tokens // T_BLK
    return pl.pallas_call(
        _tc_idx_body,
        grid=(g,),
        in_specs=[
            pl.BlockSpec((T_BLK, DIM_EMBED), lambda i: (i, 0)),
            pl.BlockSpec((HEADS, DIM_EMBED, DIM), lambda i: (0, 0, 0)),
            pl.BlockSpec((HEADS, K, DIM), lambda i: (0, 0, 0)),
            pl.BlockSpec((HEADS, K), lambda i: (0, 0)),
        ],
        out_specs=pl.BlockSpec((1, HEADS, T_BLK), lambda i: (i, 0, 0)),
        out_shape=jax.ShapeDtypeStruct((g, HEADS, T_BLK), jnp.int32),
    )(xf, rand_proj, codebook, cbn)


def _make_sc_gather_mean(tokens):
    tok_per_w = tokens // SC_WORKERS
    n_chunks = tok_per_w // GATHER_CHUNK

    @functools.partial(
        pl.kernel,
        mesh=plsc.VectorSubcoreMesh(core_axis_name="c", subcore_axis_name="s"),
        compiler_params=pltpu.CompilerParams(use_tc_tiling_on_sc=False),
        out_type=jax.ShapeDtypeStruct((tokens, DIM_MEM), jnp.float32),
        scratch_types=[
            pltpu.VMEM((n_chunks, GATHER_CHUNK), jnp.int32),
            pltpu.VMEM((n_chunks, GATHER_CHUNK), jnp.int32),
            pltpu.VMEM((tok_per_w, DIM_MEM), jnp.float32),
            pltpu.VMEM((tok_per_w, DIM_MEM), jnp.float32),
            pltpu.SemaphoreType.DMA,
        ],
    )
    def sc_gather_mean(vals_hbm, idx0_hbm, idx1_hbm, out_hbm,
                       i0_v, i1_v, r0_v, r1_v, sem):
        wid = lax.axis_index("s") * SC_CORES + lax.axis_index("c")
        base = wid * tok_per_w
        ic0 = pltpu.async_copy(idx0_hbm.at[wid], i0_v, sem)
        ic1 = pltpu.async_copy(idx1_hbm.at[wid], i1_v, sem)
        ic0.wait()
        ic1.wait()
        copies = []
        for j in range(n_chunks):
            sl = pl.ds(j * GATHER_CHUNK, GATHER_CHUNK)
            copies.append(pltpu.async_copy(vals_hbm.at[i0_v.at[j]], r0_v.at[sl], sem))
            copies.append(pltpu.async_copy(vals_hbm.at[i1_v.at[j]], r1_v.at[sl], sem))
        for c in copies:
            c.wait()

        def body(t, carry):
            for u in range(4):
                for j in range(DIM_MEM // 16):
                    sl = pl.ds(j * 16, 16)
                    r0_v[t * 4 + u, sl] = (r0_v[t * 4 + u, sl]
                                           + r1_v[t * 4 + u, sl]) * 0.5
            return carry

        lax.fori_loop(0, tok_per_w // 4, body, 0)
        pltpu.sync_copy(r0_v, out_hbm.at[pl.ds(base, tok_per_w)])

    return sc_gather_mean


def kernel(x, rand_proj, values, codebook):
    xf = x.reshape(TOKENS, DIM_EMBED)
    cbn = jnp.sum(codebook * codebook, axis=-1)
    vals_flat = values.reshape(HEADS * K, DIM_MEM)
    sc_gather = _make_sc_gather_mean(T_HALF)
    sc_shape = (SC_WORKERS, T_HALF // SC_WORKERS // GATHER_CHUNK, GATHER_CHUNK)
    idx = _tc_indices(xf, rand_proj, codebook, cbn, T_HALF)
    idxh = idx.transpose(1, 0, 2).reshape((HEADS,) + sc_shape)
    out = sc_gather(vals_flat, idxh[0], idxh[1])
    return out.reshape(B, N, DIM_MEM)


# speedup vs baseline: 1.1505x; 1.0020x over previous
import functools

import jax
import jax.numpy as jnp
from jax import lax
from jax.experimental import pallas as pl
from jax.experimental.pallas import tpu as pltpu
from jax.experimental.pallas import tpu_sc as plsc

B, N, DIM_EMBED = 8, 576, 384
DIM = 32
HEADS = 2
K = 8192
DIM_MEM = 32

TOKENS = B * N
N_SPLITS = 1
T_HALF = TOKENS // N_SPLITS
T_BLK = 384

SC_CORES = 2
SC_SUBCORES = 16
SC_WORKERS = SC_CORES * SC_SUBCORES
GATHER_CHUNK = 72


def _tc_idx_body(x_ref, rp_ref, cb_ref, cbn_ref, idx_ref):
    xb = x_ref[...]
    outs = []
    for h in range(HEADS):
        rp = rp_ref[h]
        cb = cb_ref[h]
        xp = jnp.dot(xb, rp)
        xp2 = jnp.sum(xp * xp, axis=-1, keepdims=True)
        dotn = lax.dot_general(xp * -2.0, cb, (((1,), (1,)), ((), ())))
        d2 = xp2 + dotn + cbn_ref[h][None, :]
        am = jnp.argmin(d2, axis=-1).astype(jnp.int32)
        outs.append(am + h * K)
    idx_ref[...] = jnp.stack(outs)[None]


def _tc_indices(xf, rand_proj, codebook, cbn, tokens):
    g = tokens // T_BLK
    return pl.pallas_call(
        _tc_idx_body,
        grid=(g,),
        in_specs=[
            pl.BlockSpec((T_BLK, DIM_EMBED), lambda i: (i, 0)),
            pl.BlockSpec((HEADS, DIM_EMBED, DIM), lambda i: (0, 0, 0)),
            pl.BlockSpec((HEADS, K, DIM), lambda i: (0, 0, 0)),
            pl.BlockSpec((HEADS, K), lambda i: (0, 0)),
        ],
        out_specs=pl.BlockSpec((1, HEADS, T_BLK), lambda i: (i, 0, 0)),
        out_shape=jax.ShapeDtypeStruct((g, HEADS, T_BLK), jnp.int32),
    )(xf, rand_proj, codebook, cbn)


def _make_sc_gather_mean(tokens):
    tok_per_w = tokens // SC_WORKERS
    n_chunks = tok_per_w // GATHER_CHUNK

    @functools.partial(
        pl.kernel,
        mesh=plsc.VectorSubcoreMesh(core_axis_name="c", subcore_axis_name="s"),
        compiler_params=pltpu.CompilerParams(use_tc_tiling_on_sc=False),
        out_type=jax.ShapeDtypeStruct((tokens, DIM_MEM), jnp.float32),
        scratch_types=[
            pltpu.VMEM((n_chunks, GATHER_CHUNK), jnp.int32),
            pltpu.VMEM((n_chunks, GATHER_CHUNK), jnp.int32),
            pltpu.VMEM((tok_per_w, DIM_MEM), jnp.float32),
            pltpu.VMEM((tok_per_w, DIM_MEM), jnp.float32),
            pltpu.SemaphoreType.DMA,
        ],
    )
    def sc_gather_mean(vals_hbm, idx0_hbm, idx1_hbm, out_hbm,
                       i0_v, i1_v, r0_v, r1_v, sem):
        wid = lax.axis_index("s") * SC_CORES + lax.axis_index("c")
        base = wid * tok_per_w
        ic0 = pltpu.async_copy(idx0_hbm.at[wid], i0_v, sem)
        ic1 = pltpu.async_copy(idx1_hbm.at[wid], i1_v, sem)
        ic0.wait()
        ic1.wait()
        copies = []
        for j in range(n_chunks):
            sl = pl.ds(j * GATHER_CHUNK, GATHER_CHUNK)
            copies.append(pltpu.async_copy(vals_hbm.at[i0_v.at[j]], r0_v.at[sl], sem))
            copies.append(pltpu.async_copy(vals_hbm.at[i1_v.at[j]], r1_v.at[sl], sem))
        for c in copies:
            c.wait()

        def body(t, carry):
            for u in range(4):
                for j in range(DIM_MEM // 16):
                    sl = pl.ds(j * 16, 16)
                    r0_v[t * 4 + u, sl] = (r0_v[t * 4 + u, sl]
                                           + r1_v[t * 4 + u, sl]) * 0.5
            return carry

        lax.fori_loop(0, tok_per_w // 4, body, 0)
        pltpu.sync_copy(r0_v, out_hbm.at[pl.ds(base, tok_per_w)])

    return sc_gather_mean


def kernel(x, rand_proj, values, codebook):
    xf = x.reshape(TOKENS, DIM_EMBED)
    cbn = jnp.sum(codebook * codebook, axis=-1)
    vals_flat = values.reshape(HEADS * K, DIM_MEM)
    sc_gather = _make_sc_gather_mean(T_HALF)
    sc_shape = (SC_WORKERS, T_HALF // SC_WORKERS // GATHER_CHUNK, GATHER_CHUNK)
    idx = _tc_indices(xf, rand_proj, codebook, cbn, T_HALF)
    idxh = idx.transpose(1, 0, 2).reshape((HEADS,) + sc_shape)
    out = sc_gather(vals_flat, idxh[0], idxh[1])
    return out.reshape(B, N, DIM_MEM)


# speedup vs baseline: 1.1523x; 1.0015x over previous
import functools

import jax
import jax.numpy as jnp
from jax import lax
from jax.experimental import pallas as pl
from jax.experimental.pallas import tpu as pltpu
from jax.experimental.pallas import tpu_sc as plsc

B, N, DIM_EMBED = 8, 576, 384
DIM = 32
HEADS = 2
K = 8192
DIM_MEM = 32

TOKENS = B * N
T_BLK = 384

SC_CORES = 2
SC_SUBCORES = 16
SC_WORKERS = SC_CORES * SC_SUBCORES
GATHER_CHUNK = 72


def _tc_idx_body(x_ref, rp_ref, cb_ref, cbn_ref, idx_ref):
    xb = x_ref[...]
    outs = []
    for h in range(HEADS):
        rp = rp_ref[h]
        cb = cb_ref[h]
        xp = jnp.dot(xb, rp)
        xp2 = jnp.sum(xp * xp, axis=-1, keepdims=True)
        dotn = lax.dot_general(xp * -2.0, cb, (((1,), (1,)), ((), ())))
        d2 = xp2 + dotn + cbn_ref[h][None, :]
        am = jnp.argmin(d2, axis=-1).astype(jnp.int32)
        outs.append(am + h * K)
    idx_ref[...] = jnp.stack(outs)[None]


def _tc_indices(xf, rand_proj, codebook, cbn, tokens):
    g = tokens // T_BLK
    return pl.pallas_call(
        _tc_idx_body,
        grid=(g,),
        in_specs=[
            pl.BlockSpec((T_BLK, DIM_EMBED), lambda i: (i, 0)),
            pl.BlockSpec((HEADS, DIM_EMBED, DIM), lambda i: (0, 0, 0)),
            pl.BlockSpec((HEADS, K, DIM), lambda i: (0, 0, 0)),
            pl.BlockSpec((HEADS, K), lambda i: (0, 0)),
        ],
        out_specs=pl.BlockSpec((1, HEADS, T_BLK), lambda i: (i, 0, 0)),
        out_shape=jax.ShapeDtypeStruct((g, HEADS, T_BLK), jnp.int32),
    )(xf, rand_proj, codebook, cbn)


def _make_sc_gather_mean(tokens):
    tok_per_w = tokens // SC_WORKERS
    n_chunks = tok_per_w // GATHER_CHUNK

    @functools.partial(
        pl.kernel,
        mesh=plsc.VectorSubcoreMesh(core_axis_name="c", subcore_axis_name="s"),
        compiler_params=pltpu.CompilerParams(use_tc_tiling_on_sc=False),
        out_type=jax.ShapeDtypeStruct((tokens, DIM_MEM), jnp.float32),
        scratch_types=[
            pltpu.VMEM((n_chunks, GATHER_CHUNK), jnp.int32),
            pltpu.VMEM((n_chunks, GATHER_CHUNK), jnp.int32),
            pltpu.VMEM((tok_per_w, DIM_MEM), jnp.float32),
            pltpu.VMEM((tok_per_w, DIM_MEM), jnp.float32),
            pltpu.SemaphoreType.DMA,
        ],
    )
    def sc_gather_mean(vals_hbm, idx0_hbm, idx1_hbm, out_hbm,
                       i0_v, i1_v, r0_v, r1_v, sem):
        wid = lax.axis_index("s") * SC_CORES + lax.axis_index("c")
        base = wid * tok_per_w
        ic0 = pltpu.async_copy(idx0_hbm.at[wid], i0_v, sem)
        ic1 = pltpu.async_copy(idx1_hbm.at[wid], i1_v, sem)
        ic0.wait()
        ic1.wait()
        copies = []
        for j in range(n_chunks):
            sl = pl.ds(j * GATHER_CHUNK, GATHER_CHUNK)
            copies.append(pltpu.async_copy(vals_hbm.at[i0_v.at[j]], r0_v.at[sl], sem))
            copies.append(pltpu.async_copy(vals_hbm.at[i1_v.at[j]], r1_v.at[sl], sem))
        for c in copies:
            c.wait()

        def body(t, carry):
            for u in range(4):
                for j in range(DIM_MEM // 16):
                    sl = pl.ds(j * 16, 16)
                    r0_v[t * 4 + u, sl] = (r0_v[t * 4 + u, sl]
                                           + r1_v[t * 4 + u, sl]) * 0.5
            return carry

        lax.fori_loop(0, tok_per_w // 4, body, 0)
        pltpu.sync_copy(r0_v, out_hbm.at[pl.ds(base, tok_per_w)])

    return sc_gather_mean


def kernel(x, rand_proj, values, codebook):
    xf = x.reshape(TOKENS, DIM_EMBED)
    cbn = jnp.sum(codebook * codebook, axis=-1)
    vals_flat = values.reshape(HEADS * K, DIM_MEM)
    sc_gather = _make_sc_gather_mean(TOKENS)
    sc_shape = (SC_WORKERS, TOKENS // SC_WORKERS // GATHER_CHUNK, GATHER_CHUNK)
    idx = _tc_indices(xf, rand_proj, codebook, cbn, TOKENS)
    idxh = idx.transpose(1, 0, 2).reshape((HEADS,) + sc_shape)
    out = sc_gather(vals_flat, idxh[0], idxh[1])
    return out.reshape(B, N, DIM_MEM)
